# baseline (speedup 1.0000x reference)
import functools

import jax
import jax.numpy as jnp
from jax import lax
from jax.experimental import pallas as pl
from jax.experimental.pallas import tpu as pltpu
from jax.experimental.pallas import tpu_sc as plsc

_D = 64
_SCALE = 8.0
_NC, _NS = 2, 16
_NW = _NC * _NS
_SB = 25
_NBB = 32
_TILES = _SB * _NBB
_TPW = _TILES // _NW
_UPT = 8
_U = 128
_UPW = _TPW * _UPT


@jax.jit
def _lookup(xs, table):
    mesh = plsc.VectorSubcoreMesh(core_axis_name="c", subcore_axis_name="s")

    @functools.partial(
        pl.kernel,
        out_type=jax.ShapeDtypeStruct((200, 8, _NBB, 8, 128), jnp.float32),
        mesh=mesh,
        scratch_types=[
            pltpu.VMEM((_UPW * _U,), jnp.int32),
            pltpu.VMEM((2, _U, _D), jnp.float32),
            pltpu.VMEM((2, _D, _U), jnp.float32),
            pltpu.SemaphoreType.DMA,
            pltpu.SemaphoreType.DMA,
            pltpu.SemaphoreType.DMA,
            pltpu.SemaphoreType.DMA,
        ],
        compiler_params=pltpu.CompilerParams(
            use_tc_tiling_on_sc=False, needs_layout_passes=False
        ),
    )
    def emb(xs_hbm, table_hbm, out_hbm, xb, gb, tb, g0, g1, s0, s1):
        wid = lax.axis_index("s") * _NC + lax.axis_index("c")
        t_base = wid * _TPW
        gsem = (g0, g1)
        ssem = (s0, s1)

        def start_gather(u, slot):
            pltpu.make_async_copy(
                table_hbm.at[xb.at[pl.ds(u * _U, _U)]],
                gb.at[slot],
                gsem[slot],
            ).start()

        def wait_gather(slot):
            pltpu.make_async_copy(
                table_hbm.at[xb.at[pl.ds(0, _U)]], gb.at[slot], gsem[slot]
            ).wait()

        def transpose_scale(slot):
            g = gb.at[slot]
            o = tb.at[slot]
            for grp in range(_U // 16):
                ivec = lax.iota(jnp.int32, 16) + (grp * 16)

                @pl.loop(0, _D, unroll=4)
                def _(k):
                    jvec = jnp.full((16,), k, jnp.int32)
                    val = plsc.load_gather(g, [ivec, jvec])
                    o[k, pl.ds(grp * 16, 16)] = val * _SCALE

        def start_scatter(tl, sr, slot):
            t = t_base + tl
            s = (t // _NBB) * 8 + sr
            bb = lax.rem(t, _NBB)
            for db in range(8):
                pltpu.make_async_copy(
                    tb.at[slot].at[pl.ds(db * 8, 8)],
                    out_hbm.at[s, db, bb],
                    ssem[slot],
                ).start()

        def wait_scatter(slot):
            for db in range(8):
                pltpu.make_async_copy(
                    tb.at[slot].at[pl.ds(db * 8, 8)],
                    out_hbm.at[0, db, 0],
                    ssem[slot],
                ).wait()

        pltpu.sync_copy(xs_hbm.at[pl.ds(t_base * (_UPT * _U), _UPW * _U)], xb)
        start_gather(0, 0)

        @pl.loop(0, _TPW)
        def _(tl):
            for sr in range(_UPT):
                slot = sr % 2
                nslot = (sr + 1) % 2
                if sr < _UPT - 1:
                    start_gather(tl * _UPT + sr + 1, nslot)
                else:

                    @pl.when(tl < _TPW - 1)
                    def _():
                        start_gather((tl + 1) * _UPT, nslot)

                if sr >= 2:
                    wait_scatter(slot)
                else:

                    @pl.when(tl >= 1)
                    def _():
                        wait_scatter(slot)

                wait_gather(slot)
                transpose_scale(slot)
                start_scatter(tl, sr, slot)

        wait_scatter(0)
        wait_scatter(1)

    return emb(xs, table)


def kernel(x, table):
    xs = x.astype(jnp.int32).reshape(_NBB, 128, _SB, 8).transpose(2, 0, 3, 1)
    out5 = _lookup(xs.reshape(-1), table)
    return out5.transpose(2, 4, 0, 1, 3).reshape(4096, 200, _D)


# speedup vs baseline: 1.4534x; 1.4534x over previous
import functools

import jax
import jax.numpy as jnp
from jax import lax
from jax.experimental import pallas as pl
from jax.experimental.pallas import tpu as pltpu
from jax.experimental.pallas import tpu_sc as plsc

_D = 64
_SCALE = 8.0
_NC, _NS = 2, 16
_NW = _NC * _NS
_SB = 25
_NBB = 32
_TILES = _SB * _NBB
_TPW = _TILES // _NW
_UPT = 8
_U = 128
_UPW = _TPW * _UPT


@jax.jit
def _lookup(xs, table):
    mesh = plsc.VectorSubcoreMesh(core_axis_name="c", subcore_axis_name="s")

    @functools.partial(
        pl.kernel,
        out_type=jax.ShapeDtypeStruct((200, 8, _NBB, 8, 128), jnp.float32),
        mesh=mesh,
        scratch_types=[
            pltpu.VMEM((_UPW * _U,), jnp.int32),
            pltpu.VMEM((2, _U, _D), jnp.float32),
            pltpu.VMEM((2, _D, _U), jnp.float32),
            pltpu.SemaphoreType.DMA,
            pltpu.SemaphoreType.DMA,
            pltpu.SemaphoreType.DMA,
            pltpu.SemaphoreType.DMA,
        ],
        compiler_params=pltpu.CompilerParams(
            use_tc_tiling_on_sc=False, needs_layout_passes=False
        ),
    )
    def emb(xs_hbm, table_hbm, out_hbm, xb, gb, tb, g0, g1, s0, s1):
        wid = lax.axis_index("s") * _NC + lax.axis_index("c")
        t_base = wid * _TPW
        gsem = (g0, g1)
        ssem = (s0, s1)

        def start_gather(u, slot):
            pltpu.make_async_copy(
                table_hbm.at[xb.at[pl.ds(u * _U, _U)]],
                gb.at[slot],
                gsem[slot],
            ).start()

        def wait_gather(slot):
            pltpu.make_async_copy(
                table_hbm.at[xb.at[pl.ds(0, _U)]], gb.at[slot], gsem[slot]
            ).wait()

        def transpose_scale(slot):
            g = gb.at[slot]
            o = tb.at[slot]
            for grp in range(_U // 16):
                ivec = lax.iota(jnp.int32, 16) + (grp * 16)

                @plsc.parallel_loop(0, _D, unroll=8)
                def _(k):
                    jvec = jnp.full((16,), k, jnp.int32)
                    val = plsc.load_gather(g, [ivec, jvec])
                    o[k, pl.ds(grp * 16, 16)] = val * _SCALE

        def start_scatter(tl, sr, slot):
            t = t_base + tl
            s = (t // _NBB) * 8 + sr
            bb = lax.rem(t, _NBB)
            for db in range(8):
                pltpu.make_async_copy(
                    tb.at[slot].at[pl.ds(db * 8, 8)],
                    out_hbm.at[s, db, bb],
                    ssem[slot],
                ).start()

        def wait_scatter(slot):
            for db in range(8):
                pltpu.make_async_copy(
                    tb.at[slot].at[pl.ds(db * 8, 8)],
                    out_hbm.at[0, db, 0],
                    ssem[slot],
                ).wait()

        pltpu.sync_copy(xs_hbm.at[pl.ds(t_base * (_UPT * _U), _UPW * _U)], xb)
        start_gather(0, 0)

        @pl.loop(0, _TPW)
        def _(tl):
            for sr in range(_UPT):
                slot = sr % 2
                nslot = (sr + 1) % 2
                if sr < _UPT - 1:
                    start_gather(tl * _UPT + sr + 1, nslot)
                else:

                    @pl.when(tl < _TPW - 1)
                    def _():
                        start_gather((tl + 1) * _UPT, nslot)

                if sr >= 2:
                    wait_scatter(slot)
                else:

                    @pl.when(tl >= 1)
                    def _():
                        wait_scatter(slot)

                wait_gather(slot)
                transpose_scale(slot)
                start_scatter(tl, sr, slot)

        wait_scatter(0)
        wait_scatter(1)

    return emb(xs, table)


def kernel(x, table):
    xs = x.astype(jnp.int32).reshape(_NBB, 128, _SB, 8).transpose(2, 0, 3, 1)
    out5 = _lookup(xs.reshape(-1), table)
    return out5.transpose(2, 4, 0, 1, 3).reshape(4096, 200, _D)
